# baseline (speedup 1.0000x reference)
import jax
import jax.numpy as jnp
from jax import lax
from jax.experimental import pallas as pl
from jax.experimental.pallas import tpu as pltpu
from jax.experimental.pallas import tpu_sc as plsc

_B = 100
_ROWS = 1000
_D = 128
_L = 16
_NC = 2
_NS = 16
_CHUNKS = _D // _L
_GGRP = (_NC * _NS) // _CHUNKS
_GPW = _B // _GGRP


def _rsqrt_sc(x):
    i = lax.bitcast_convert_type(x, jnp.int32)
    i = 0x5F3759DF - lax.shift_right_logical(i, 1)
    y = lax.bitcast_convert_type(i, jnp.float32)
    for _ in range(3):
        y = y * (1.5 - 0.5 * x * y * y)
    return y


def _sc_body(x_hbm, w_hbm, b_hbm, ms_hbm, o_hbm, panel, wv, bv, msv):
    wid = lax.axis_index("s") * _NC + lax.axis_index("c")
    chunk = lax.rem(wid, _CHUNKS)
    col = chunk * _L
    g0 = lax.div(wid, _CHUNKS) * _GPW
    pltpu.sync_copy(w_hbm, wv)
    pltpu.sync_copy(b_hbm, bv)
    pltpu.sync_copy(ms_hbm, msv)
    wc = wv[pl.ds(col, _L)]
    bc = bv[pl.ds(col, _L)]
    msc = msv[pl.ds(col, _L)]
    inv_n = 1.0 / _ROWS
    for t in range(_GPW):
        g = g0 + t
        pltpu.sync_copy(x_hbm.at[g, :, pl.ds(col, _L)], panel)

        def acc(r, carry):
            s1, s2 = carry
            v = panel[r, :]
            return s1 + v, s2 + v * v

        zeros = jnp.zeros((_L,), jnp.float32)
        s1, s2 = lax.fori_loop(0, _ROWS, acc, (zeros, zeros))
        m = s1 * inv_n
        mm = m * msc
        var = s2 * inv_n - 2.0 * mm * m + mm * mm
        scale = wc * _rsqrt_sc(var + 1e-6)
        shift = bc - mm * scale

        def norm(r, carry):
            panel[r, :] = panel[r, :] * scale + shift
            return carry

        lax.fori_loop(0, _ROWS, norm, 0)
        pltpu.sync_copy(panel, o_hbm.at[g, :, pl.ds(col, _L)])


def kernel(tensor, weight, bias, mean_scale, batch_num_nodes):
    n, d = tensor.shape
    b = batch_num_nodes.shape[0]
    rows = n // b
    x3 = tensor.reshape(b, rows, d)
    mesh = plsc.VectorSubcoreMesh(core_axis_name="c", subcore_axis_name="s")
    run = pl.kernel(
        _sc_body,
        mesh=mesh,
        out_type=jax.ShapeDtypeStruct((b, rows, d), tensor.dtype),
        scratch_types=[
            pltpu.VMEM((rows, _L), jnp.float32),
            pltpu.VMEM((d,), jnp.float32),
            pltpu.VMEM((d,), jnp.float32),
            pltpu.VMEM((d,), jnp.float32),
        ],
        compiler_params=pltpu.CompilerParams(use_tc_tiling_on_sc=False),
    )
    out = run(x3, weight, bias, mean_scale)
    return out.reshape(n, d)


# speedup vs baseline: 2.1586x; 2.1586x over previous
import jax
import jax.numpy as jnp
from jax import lax
from jax.experimental import pallas as pl
from jax.experimental.pallas import tpu as pltpu
from jax.experimental.pallas import tpu_sc as plsc

_B = 100
_ROWS = 1000
_D = 128
_L = 16
_NC = 2
_NS = 16
_CHUNKS = _D // _L
_GGRP = (_NC * _NS) // _CHUNKS
_GPW = _B // _GGRP


def _rsqrt_sc(x):
    i = lax.bitcast_convert_type(x, jnp.int32)
    i = 0x5F3759DF - lax.shift_right_logical(i, 1)
    y = lax.bitcast_convert_type(i, jnp.float32)
    for _ in range(3):
        y = y * (1.5 - 0.5 * x * y * y)
    return y


def _sc_body(x_hbm, w_hbm, b_hbm, ms_hbm, o_hbm, panel, wv, bv, msv):
    wid = lax.axis_index("s") * _NC + lax.axis_index("c")
    chunk = lax.rem(wid, _CHUNKS)
    col = chunk * _L
    g0 = lax.div(wid, _CHUNKS) * _GPW
    pltpu.sync_copy(w_hbm, wv)
    pltpu.sync_copy(b_hbm, bv)
    pltpu.sync_copy(ms_hbm, msv)
    wc = wv[pl.ds(col, _L)]
    bc = bv[pl.ds(col, _L)]
    msc = msv[pl.ds(col, _L)]
    inv_n = 1.0 / _ROWS
    for t in range(_GPW):
        g = g0 + t
        pltpu.sync_copy(x_hbm.at[g, :, pl.ds(col, _L)], panel)

        def acc(i, carry):
            a1, a2, b1, b2, c1, c2, d1, d2 = carry
            r = i * 4
            va = panel[r, :]
            vb = panel[r + 1, :]
            vc = panel[r + 2, :]
            vd = panel[r + 3, :]
            return (a1 + va, a2 + va * va, b1 + vb, b2 + vb * vb,
                    c1 + vc, c2 + vc * vc, d1 + vd, d2 + vd * vd)

        z = jnp.zeros((_L,), jnp.float32)
        a1, a2, b1, b2, c1, c2, d1, d2 = lax.fori_loop(
            0, _ROWS // 4, acc, (z, z, z, z, z, z, z, z))
        s1 = (a1 + b1) + (c1 + d1)
        s2 = (a2 + b2) + (c2 + d2)
        m = s1 * inv_n
        mm = m * msc
        var = s2 * inv_n - 2.0 * mm * m + mm * mm
        scale = wc * _rsqrt_sc(var + 1e-6)
        shift = bc - mm * scale

        def norm(i, carry):
            r = i * 4
            panel[r, :] = panel[r, :] * scale + shift
            panel[r + 1, :] = panel[r + 1, :] * scale + shift
            panel[r + 2, :] = panel[r + 2, :] * scale + shift
            panel[r + 3, :] = panel[r + 3, :] * scale + shift
            return carry

        lax.fori_loop(0, _ROWS // 4, norm, 0)
        pltpu.sync_copy(panel, o_hbm.at[g, :, pl.ds(col, _L)])


def kernel(tensor, weight, bias, mean_scale, batch_num_nodes):
    n, d = tensor.shape
    b = batch_num_nodes.shape[0]
    rows = n // b
    x3 = tensor.reshape(b, rows, d)
    mesh = plsc.VectorSubcoreMesh(core_axis_name="c", subcore_axis_name="s")
    run = pl.kernel(
        _sc_body,
        mesh=mesh,
        out_type=jax.ShapeDtypeStruct((b, rows, d), tensor.dtype),
        scratch_types=[
            pltpu.VMEM((rows, _L), jnp.float32),
            pltpu.VMEM((d,), jnp.float32),
            pltpu.VMEM((d,), jnp.float32),
            pltpu.VMEM((d,), jnp.float32),
        ],
        compiler_params=pltpu.CompilerParams(use_tc_tiling_on_sc=False),
    )
    out = run(x3, weight, bias, mean_scale)
    return out.reshape(n, d)


# speedup vs baseline: 2.9862x; 1.3834x over previous
import jax
import jax.numpy as jnp
from jax import lax
from jax.experimental import pallas as pl
from jax.experimental.pallas import tpu as pltpu
from jax.experimental.pallas import tpu_sc as plsc

_B = 100
_ROWS = 1000
_D = 128
_L = 16
_NC = 2
_NS = 16
_CHUNKS = _D // _L
_GGRP = (_NC * _NS) // _CHUNKS
_GPW = _B // _GGRP


def _rsqrt_sc(x):
    i = lax.bitcast_convert_type(x, jnp.int32)
    i = 0x5F3759DF - lax.shift_right_logical(i, 1)
    y = lax.bitcast_convert_type(i, jnp.float32)
    for _ in range(3):
        y = y * (1.5 - 0.5 * x * y * y)
    return y


def _sc_body(x_hbm, w_hbm, b_hbm, ms_hbm, o_hbm, p0, p1, wv, bv, msv,
             si0, si1, so0, so1):
    wid = lax.axis_index("s") * _NC + lax.axis_index("c")
    chunk = lax.rem(wid, _CHUNKS)
    col = chunk * _L
    g0 = lax.div(wid, _CHUNKS) * _GPW
    pltpu.sync_copy(w_hbm, wv)
    pltpu.sync_copy(b_hbm, bv)
    pltpu.sync_copy(ms_hbm, msv)
    wc = wv[pl.ds(col, _L)]
    bc = bv[pl.ds(col, _L)]
    msc = msv[pl.ds(col, _L)]
    inv_n = 1.0 / _ROWS
    panels = (p0, p1)
    isems = (si0, si1)
    osems = (so0, so1)
    h_in = [None, None]
    h_out = [None, None]
    h_in[0] = pltpu.async_copy(
        x_hbm.at[g0, :, pl.ds(col, _L)], panels[0], isems[0])
    for t in range(_GPW):
        k = t % 2
        nk = (t + 1) % 2
        if t + 1 < _GPW:
            if h_out[nk] is not None:
                h_out[nk].wait()
            h_in[nk] = pltpu.async_copy(
                x_hbm.at[g0 + t + 1, :, pl.ds(col, _L)], panels[nk], isems[nk])
        h_in[k].wait()
        panel = panels[k]

        def acc(i, carry):
            a1, a2, b1, b2, c1, c2, d1, d2 = carry
            r = i * 4
            va = panel[r, :]
            vb = panel[r + 1, :]
            vc = panel[r + 2, :]
            vd = panel[r + 3, :]
            return (a1 + va, a2 + va * va, b1 + vb, b2 + vb * vb,
                    c1 + vc, c2 + vc * vc, d1 + vd, d2 + vd * vd)

        z = jnp.zeros((_L,), jnp.float32)
        a1, a2, b1, b2, c1, c2, d1, d2 = lax.fori_loop(
            0, _ROWS // 4, acc, (z, z, z, z, z, z, z, z))
        s1 = (a1 + b1) + (c1 + d1)
        s2 = (a2 + b2) + (c2 + d2)
        m = s1 * inv_n
        mm = m * msc
        var = s2 * inv_n - 2.0 * mm * m + mm * mm
        scale = wc * _rsqrt_sc(var + 1e-6)
        shift = bc - mm * scale

        def norm(i, carry):
            r = i * 4
            panel[r, :] = panel[r, :] * scale + shift
            panel[r + 1, :] = panel[r + 1, :] * scale + shift
            panel[r + 2, :] = panel[r + 2, :] * scale + shift
            panel[r + 3, :] = panel[r + 3, :] * scale + shift
            return carry

        lax.fori_loop(0, _ROWS // 4, norm, 0)
        h_out[k] = pltpu.async_copy(
            panel, o_hbm.at[g0 + t, :, pl.ds(col, _L)], osems[k])
    for h in h_out:
        if h is not None:
            h.wait()


def kernel(tensor, weight, bias, mean_scale, batch_num_nodes):
    n, d = tensor.shape
    b = batch_num_nodes.shape[0]
    rows = n // b
    x3 = tensor.reshape(b, rows, d)
    mesh = plsc.VectorSubcoreMesh(core_axis_name="c", subcore_axis_name="s")
    run = pl.kernel(
        _sc_body,
        mesh=mesh,
        out_type=jax.ShapeDtypeStruct((b, rows, d), tensor.dtype),
        scratch_types=[
            pltpu.VMEM((rows, _L), jnp.float32),
            pltpu.VMEM((rows, _L), jnp.float32),
            pltpu.VMEM((d,), jnp.float32),
            pltpu.VMEM((d,), jnp.float32),
            pltpu.VMEM((d,), jnp.float32),
            pltpu.SemaphoreType.DMA,
            pltpu.SemaphoreType.DMA,
            pltpu.SemaphoreType.DMA,
            pltpu.SemaphoreType.DMA,
        ],
        compiler_params=pltpu.CompilerParams(use_tc_tiling_on_sc=False),
    )
    out = run(x3, weight, bias, mean_scale)
    return out.reshape(n, d)
